# baseline (speedup 1.0000x reference)
import jax
import jax.numpy as jnp
from jax import lax
from jax.experimental import pallas as pl
from jax.experimental.pallas import tpu as pltpu
from jax.experimental.pallas import tpu_sc as plsc

N = 1 << 24
K = 1 << 23
NC, NS, L = 2, 16, 16

RE = 1 << 20
EP = RE + 2048
PSL = EP // NS

PER_T = N // NS
CHUNK = 8192
NCHUNK = PER_T // CHUNK
Q = 4608
ZC = Q


def _sc_body(x_hbm, out_hbm, pres, xbA, xbB, ixA, ixB, vlA, vlB,
             insA, insB, scsA, scsB):
    c = lax.axis_index("c")
    s = lax.axis_index("s")
    zbuf = vlA

    def fill(i, _):
        zbuf[pl.ds(i * L, L)] = jnp.zeros((L,), jnp.int32)
        return 0
    lax.fori_loop(0, ZC // L, fill, 0)

    zbase = s * PSL

    def zero(i, _):
        pltpu.sync_copy(zbuf, pres.at[pl.ds(zbase + i * ZC, ZC)])
        return 0
    lax.fori_loop(0, PSL // ZC, zero, 0)
    ztail = PSL % ZC
    pltpu.sync_copy(zbuf.at[pl.ds(0, ztail)],
                    pres.at[pl.ds(zbase + (PSL // ZC) * ZC, ztail)])

    def fill_ix(i, _):
        dv = jnp.full((L,), RE, jnp.int32)
        ixA[pl.ds(i * L, L)] = dv
        ixB[pl.ds(i * L, L)] = dv
        return 0
    lax.fori_loop(0, Q // L, fill_ix, 0)

    plsc.subcore_barrier()

    base_c = c * (4 * RE)
    re_u = jnp.uint32(RE)

    def start_in(g, xb, sem):
        pltpu.make_async_copy(
            x_hbm.at[pl.ds(s * PER_T + g * CHUNK, CHUNK)], xb, sem).start()

    def wait_in(xb, sem):
        pltpu.make_async_copy(x_hbm.at[pl.ds(0, CHUNK)], xb, sem).wait()

    padi = lax.iota(jnp.int32, L) + 1
    zeros16 = jnp.zeros((L,), jnp.int32)
    qv = jnp.int32(Q)

    def compute(xb, ix, vl):
        def vec(r, cur):
            for u in range(128 // L):
                o = r * 128 + u * L
                xv = xb[pl.ds(o, L)]
                kv = (xv * float(K)).astype(jnp.int32)
                off = kv - base_c
                offu = off.astype(jnp.uint32)
                word = lax.shift_right_logical(offu, jnp.uint32(2))
                m = word < re_u
                pos = cur + plsc.cumsum(m.astype(jnp.int32))
                wi = word.astype(jnp.int32)
                b8 = jnp.left_shift(off & 3, 3)
                val = jnp.left_shift(jnp.int32(1), b8)
                plsc.store_scatter(ix, [pos], wi, mask=m)
                plsc.store_scatter(vl, [pos], val, mask=m)
                cur = cur + plsc.all_reduce_population_count(m)
            return cur
        cur = lax.fori_loop(0, CHUNK // 128, vec,
                            jnp.full((L,), -1, jnp.int32))

        def pad(j, _):
            p = cur + padi + j * L
            plsc.store_scatter(vl, [p], zeros16, mask=p < qv)
            return 0
        lax.fori_loop(0, 64, pad, 0)

    def start_scatter(ix, vl, sem):
        pltpu.async_copy(vl, pres.at[ix], sem, add=True)

    def wait_scatter(ix, vl, sem):
        pltpu.make_async_copy(vl, pres.at[ix], sem).wait()

    start_in(0, xbA, insA)
    start_in(1, xbB, insB)

    def pbody(p, _):
        g0 = 2 * p
        wait_in(xbA, insA)

        @pl.when(p > 0)
        def _():
            wait_scatter(ixA, vlA, scsA)
        compute(xbA, ixA, vlA)

        @pl.when(g0 + 2 < NCHUNK)
        def _():
            start_in(g0 + 2, xbA, insA)
        start_scatter(ixA, vlA, scsA)

        wait_in(xbB, insB)

        @pl.when(p > 0)
        def _():
            wait_scatter(ixB, vlB, scsB)
        compute(xbB, ixB, vlB)

        @pl.when(g0 + 3 < NCHUNK)
        def _():
            start_in(g0 + 3, xbB, insB)
        start_scatter(ixB, vlB, scsB)
        return 0
    lax.fori_loop(0, NCHUNK // 2, pbody, 0)

    wait_scatter(ixA, vlA, scsA)
    wait_scatter(ixB, vlB, scsB)

    plsc.subcore_barrier()

    size_full = PSL
    size_last = PSL - 2048

    @pl.when(s < NS - 1)
    def _():
        pltpu.sync_copy(pres.at[pl.ds(zbase, size_full)],
                        out_hbm.at[c, pl.ds(zbase, size_full)])

    @pl.when(s == NS - 1)
    def _():
        pltpu.sync_copy(pres.at[pl.ds(zbase, size_last)],
                        out_hbm.at[c, pl.ds(zbase, size_last)])


def _sc_scatter(x):
    mesh = plsc.VectorSubcoreMesh(core_axis_name="c", subcore_axis_name="s")
    return pl.kernel(
        _sc_body,
        out_type=jax.ShapeDtypeStruct((NC, RE), jnp.int32),
        mesh=mesh,
        compiler_params=pltpu.CompilerParams(needs_layout_passes=False),
        scratch_types=[
            pltpu.VMEM_SHARED((EP,), jnp.int32),
            pltpu.VMEM((CHUNK,), jnp.float32),
            pltpu.VMEM((CHUNK,), jnp.float32),
            pltpu.VMEM((Q,), jnp.int32),
            pltpu.VMEM((Q,), jnp.int32),
            pltpu.VMEM((Q,), jnp.int32),
            pltpu.VMEM((Q,), jnp.int32),
            pltpu.SemaphoreType.DMA,
            pltpu.SemaphoreType.DMA,
            pltpu.SemaphoreType.DMA,
            pltpu.SemaphoreType.DMA,
        ],
    )(x)


ROWS = NC * RE // 1024
BLK = 256
GRID = ROWS // BLK


def _tc_merge_body(w_ref, out_ref):
    g = pl.program_id(0)

    @pl.when(g == 0)
    def _():
        out_ref[0, 0] = 0.0

    w = w_ref[...]
    row = lax.broadcasted_iota(jnp.int32, (BLK, 1024), 0)
    col = lax.broadcasted_iota(jnp.int32, (BLK, 1024), 1)
    k0 = ((g * BLK + row) * 1024 + col) * 4
    k0f = k0.astype(jnp.float32)
    total = out_ref[0, 0]
    for b in range(4):
        mb = (lax.shift_right_logical(w, 8 * b) & 0xFF) != 0
        total = total + jnp.sum(jnp.where(mb, k0f + float(b), 0.0))
    out_ref[0, 0] = total

    @pl.when(g == GRID - 1)
    def _():
        out_ref[0, 0] = out_ref[0, 0] * (2.0 ** -23)


def _tc_merge(p):
    p2 = p.reshape(ROWS, 1024)
    out = pl.pallas_call(
        _tc_merge_body,
        grid=(GRID,),
        in_specs=[pl.BlockSpec((BLK, 1024), lambda g: (g, 0))],
        out_specs=pl.BlockSpec(memory_space=pltpu.MemorySpace.SMEM),
        out_shape=jax.ShapeDtypeStruct((1, 1), jnp.float32),
    )(p2)
    return out.reshape(())


def kernel(x):
    counts = _sc_scatter(x)
    return _tc_merge(counts)


# speedup vs baseline: 3.1686x; 3.1686x over previous
import jax
import jax.numpy as jnp
from jax import lax
from jax.experimental import pallas as pl
from jax.experimental.pallas import tpu as pltpu
from jax.experimental.pallas import tpu_sc as plsc

N = 1 << 24
K = 1 << 23
NC, NS, L = 2, 16, 16

RE = 1 << 20
EP = RE + 2048
PSL = EP // NS

PER_T = N // NS
CHUNK = 8192
NCHUNK = PER_T // CHUNK
ZC = CHUNK


def _sc_body(x_hbm, out_hbm, pres, xbA, xbB, ixA, ixB, vlA, vlB,
             insA, insB, scsA, scsB):
    c = lax.axis_index("c")
    s = lax.axis_index("s")
    zbuf = vlA

    def fill(i, _):
        zbuf[pl.ds(i * L, L)] = jnp.zeros((L,), jnp.int32)
        return 0
    lax.fori_loop(0, ZC // L, fill, 0)

    zbase = s * PSL

    def zero(i, _):
        pltpu.sync_copy(zbuf, pres.at[pl.ds(zbase + i * ZC, ZC)])
        return 0
    lax.fori_loop(0, PSL // ZC, zero, 0)
    ztail = PSL % ZC
    pltpu.sync_copy(zbuf.at[pl.ds(0, ztail)],
                    pres.at[pl.ds(zbase + (PSL // ZC) * ZC, ztail)])

    plsc.subcore_barrier()

    base_c = c * (4 * RE)
    re_u = jnp.uint32(RE)
    dmask = jnp.uint32(2047)

    def start_in(g, xb, sem):
        pltpu.make_async_copy(
            x_hbm.at[pl.ds(s * PER_T + g * CHUNK, CHUNK)], xb, sem).start()

    def wait_in(xb, sem):
        pltpu.make_async_copy(x_hbm.at[pl.ds(0, CHUNK)], xb, sem).wait()

    def compute(xb, ix, vl):
        def vec(r, _):
            for u in range(128 // L):
                o = r * 128 + u * L
                xv = xb[pl.ds(o, L)]
                kv = (xv * float(K)).astype(jnp.int32)
                off = kv - base_c
                offu = off.astype(jnp.uint32)
                word = lax.shift_right_logical(offu, jnp.uint32(2))
                idx = jnp.where(word < re_u, word, re_u + (word & dmask))
                ix[pl.ds(o, L)] = idx.astype(jnp.int32)
                b8 = jnp.left_shift(off & 3, 3)
                vl[pl.ds(o, L)] = jnp.left_shift(jnp.int32(1), b8)
            return 0
        lax.fori_loop(0, CHUNK // 128, vec, 0)

    def start_scatter(ix, vl, sem):
        pltpu.async_copy(vl, pres.at[ix], sem, add=True)

    def wait_scatter(ix, vl, sem):
        pltpu.make_async_copy(vl, pres.at[ix], sem).wait()

    start_in(0, xbA, insA)
    start_in(1, xbB, insB)

    def pbody(p, _):
        g0 = 2 * p
        wait_in(xbA, insA)

        @pl.when(p > 0)
        def _():
            wait_scatter(ixA, vlA, scsA)
        compute(xbA, ixA, vlA)

        @pl.when(g0 + 2 < NCHUNK)
        def _():
            start_in(g0 + 2, xbA, insA)
        start_scatter(ixA, vlA, scsA)

        wait_in(xbB, insB)

        @pl.when(p > 0)
        def _():
            wait_scatter(ixB, vlB, scsB)
        compute(xbB, ixB, vlB)

        @pl.when(g0 + 3 < NCHUNK)
        def _():
            start_in(g0 + 3, xbB, insB)
        start_scatter(ixB, vlB, scsB)
        return 0
    lax.fori_loop(0, NCHUNK // 2, pbody, 0)

    wait_scatter(ixA, vlA, scsA)
    wait_scatter(ixB, vlB, scsB)

    plsc.subcore_barrier()

    size_full = PSL
    size_last = PSL - 2048

    @pl.when(s < NS - 1)
    def _():
        pltpu.sync_copy(pres.at[pl.ds(zbase, size_full)],
                        out_hbm.at[c, pl.ds(zbase, size_full)])

    @pl.when(s == NS - 1)
    def _():
        pltpu.sync_copy(pres.at[pl.ds(zbase, size_last)],
                        out_hbm.at[c, pl.ds(zbase, size_last)])


def _sc_scatter(x):
    mesh = plsc.VectorSubcoreMesh(core_axis_name="c", subcore_axis_name="s")
    return pl.kernel(
        _sc_body,
        out_type=jax.ShapeDtypeStruct((NC, RE), jnp.int32),
        mesh=mesh,
        compiler_params=pltpu.CompilerParams(needs_layout_passes=False),
        scratch_types=[
            pltpu.VMEM_SHARED((EP,), jnp.int32),
            pltpu.VMEM((CHUNK,), jnp.float32),
            pltpu.VMEM((CHUNK,), jnp.float32),
            pltpu.VMEM((CHUNK,), jnp.int32),
            pltpu.VMEM((CHUNK,), jnp.int32),
            pltpu.VMEM((CHUNK,), jnp.int32),
            pltpu.VMEM((CHUNK,), jnp.int32),
            pltpu.SemaphoreType.DMA,
            pltpu.SemaphoreType.DMA,
            pltpu.SemaphoreType.DMA,
            pltpu.SemaphoreType.DMA,
        ],
    )(x)


ROWS = NC * RE // 1024
BLK = 256
GRID = ROWS // BLK


def _tc_merge_body(w_ref, out_ref):
    g = pl.program_id(0)

    @pl.when(g == 0)
    def _():
        out_ref[0, 0] = 0.0

    w = w_ref[...]
    row = lax.broadcasted_iota(jnp.int32, (BLK, 1024), 0)
    col = lax.broadcasted_iota(jnp.int32, (BLK, 1024), 1)
    k0 = ((g * BLK + row) * 1024 + col) * 4
    k0f = k0.astype(jnp.float32)
    total = out_ref[0, 0]
    for b in range(4):
        mb = (lax.shift_right_logical(w, 8 * b) & 0xFF) != 0
        total = total + jnp.sum(jnp.where(mb, k0f + float(b), 0.0))
    out_ref[0, 0] = total

    @pl.when(g == GRID - 1)
    def _():
        out_ref[0, 0] = out_ref[0, 0] * (2.0 ** -23)


def _tc_merge(p):
    p2 = p.reshape(ROWS, 1024)
    out = pl.pallas_call(
        _tc_merge_body,
        grid=(GRID,),
        in_specs=[pl.BlockSpec((BLK, 1024), lambda g: (g, 0))],
        out_specs=pl.BlockSpec(memory_space=pltpu.MemorySpace.SMEM),
        out_shape=jax.ShapeDtypeStruct((1, 1), jnp.float32),
    )(p2)
    return out.reshape(())


def kernel(x):
    counts = _sc_scatter(x)
    return _tc_merge(counts)


# speedup vs baseline: 3.1723x; 1.0012x over previous
import jax
import jax.numpy as jnp
from jax import lax
from jax.experimental import pallas as pl
from jax.experimental.pallas import tpu as pltpu
from jax.experimental.pallas import tpu_sc as plsc

N = 1 << 24
K = 1 << 23
NC, NS, L = 2, 16, 16

RE = 1 << 20
EP = RE + 2048
PSL = EP // NS

PER_T = N // NS
CHUNK = 4096
NCHUNK = PER_T // CHUNK
NSETS = 4
ZC = CHUNK


def _sc_body(x_hbm, out_hbm, pres,
             xbA, xbB, xbC, xbD, ixA, ixB, ixC, ixD, vlA, vlB, vlC, vlD,
             insA, insB, insC, insD, scsA, scsB, scsC, scsD):
    c = lax.axis_index("c")
    s = lax.axis_index("s")
    sets = ((xbA, ixA, vlA, insA, scsA), (xbB, ixB, vlB, insB, scsB),
            (xbC, ixC, vlC, insC, scsC), (xbD, ixD, vlD, insD, scsD))
    zbuf = vlA

    def fill(i, _):
        zbuf[pl.ds(i * L, L)] = jnp.zeros((L,), jnp.int32)
        return 0
    lax.fori_loop(0, ZC // L, fill, 0)

    zbase = s * PSL

    def zero(i, _):
        pltpu.sync_copy(zbuf, pres.at[pl.ds(zbase + i * ZC, ZC)])
        return 0
    lax.fori_loop(0, PSL // ZC, zero, 0)
    ztail = PSL % ZC
    pltpu.sync_copy(zbuf.at[pl.ds(0, ztail)],
                    pres.at[pl.ds(zbase + (PSL // ZC) * ZC, ztail)])

    plsc.subcore_barrier()

    base_c = c * (4 * RE)
    re_u = jnp.uint32(RE)
    dmask = jnp.uint32(2047)

    def start_in(g, xb, sem):
        pltpu.make_async_copy(
            x_hbm.at[pl.ds(s * PER_T + g * CHUNK, CHUNK)], xb, sem).start()

    def wait_in(xb, sem):
        pltpu.make_async_copy(x_hbm.at[pl.ds(0, CHUNK)], xb, sem).wait()

    def compute(xb, ix, vl):
        def vec(r, _):
            for u in range(128 // L):
                o = r * 128 + u * L
                xv = xb[pl.ds(o, L)]
                kv = (xv * float(K)).astype(jnp.int32)
                off = kv - base_c
                offu = off.astype(jnp.uint32)
                word = lax.shift_right_logical(offu, jnp.uint32(2))
                idx = jnp.where(word < re_u, word, re_u + (word & dmask))
                ix[pl.ds(o, L)] = idx.astype(jnp.int32)
                b8 = jnp.left_shift(off & 3, 3)
                vl[pl.ds(o, L)] = jnp.left_shift(jnp.int32(1), b8)
            return 0
        lax.fori_loop(0, CHUNK // 128, vec, 0)

    def start_scatter(ix, vl, sem):
        pltpu.async_copy(vl, pres.at[ix], sem, add=True)

    def wait_scatter(ix, vl, sem):
        pltpu.make_async_copy(vl, pres.at[ix], sem).wait()

    for si, (xb, ix, vl, ins, scs) in enumerate(sets):
        start_in(si, xb, ins)

    def pbody(p, _):
        g0 = NSETS * p
        for si, (xb, ix, vl, ins, scs) in enumerate(sets):
            wait_in(xb, ins)

            @pl.when(p > 0)
            def _(ix=ix, vl=vl, scs=scs):
                wait_scatter(ix, vl, scs)
            compute(xb, ix, vl)

            @pl.when(g0 + si + NSETS < NCHUNK)
            def _(g=g0 + si + NSETS, xb=xb, ins=ins):
                start_in(g, xb, ins)
            start_scatter(ix, vl, scs)
        return 0
    lax.fori_loop(0, NCHUNK // NSETS, pbody, 0)

    for si, (xb, ix, vl, ins, scs) in enumerate(sets):
        wait_scatter(ix, vl, scs)

    plsc.subcore_barrier()

    size_full = PSL
    size_last = PSL - 2048

    @pl.when(s < NS - 1)
    def _():
        pltpu.sync_copy(pres.at[pl.ds(zbase, size_full)],
                        out_hbm.at[c, pl.ds(zbase, size_full)])

    @pl.when(s == NS - 1)
    def _():
        pltpu.sync_copy(pres.at[pl.ds(zbase, size_last)],
                        out_hbm.at[c, pl.ds(zbase, size_last)])


def _sc_scatter(x):
    mesh = plsc.VectorSubcoreMesh(core_axis_name="c", subcore_axis_name="s")
    return pl.kernel(
        _sc_body,
        out_type=jax.ShapeDtypeStruct((NC, RE), jnp.int32),
        mesh=mesh,
        compiler_params=pltpu.CompilerParams(needs_layout_passes=False),
        scratch_types=[
            pltpu.VMEM_SHARED((EP,), jnp.int32),
            pltpu.VMEM((CHUNK,), jnp.float32),
            pltpu.VMEM((CHUNK,), jnp.float32),
            pltpu.VMEM((CHUNK,), jnp.float32),
            pltpu.VMEM((CHUNK,), jnp.float32),
            pltpu.VMEM((CHUNK,), jnp.int32),
            pltpu.VMEM((CHUNK,), jnp.int32),
            pltpu.VMEM((CHUNK,), jnp.int32),
            pltpu.VMEM((CHUNK,), jnp.int32),
            pltpu.VMEM((CHUNK,), jnp.int32),
            pltpu.VMEM((CHUNK,), jnp.int32),
            pltpu.VMEM((CHUNK,), jnp.int32),
            pltpu.VMEM((CHUNK,), jnp.int32),
            pltpu.SemaphoreType.DMA,
            pltpu.SemaphoreType.DMA,
            pltpu.SemaphoreType.DMA,
            pltpu.SemaphoreType.DMA,
            pltpu.SemaphoreType.DMA,
            pltpu.SemaphoreType.DMA,
            pltpu.SemaphoreType.DMA,
            pltpu.SemaphoreType.DMA,
        ],
    )(x)


ROWS = NC * RE // 1024
BLK = 256
GRID = ROWS // BLK


def _tc_merge_body(w_ref, out_ref):
    g = pl.program_id(0)

    @pl.when(g == 0)
    def _():
        out_ref[0, 0] = 0.0

    w = w_ref[...]
    row = lax.broadcasted_iota(jnp.int32, (BLK, 1024), 0)
    col = lax.broadcasted_iota(jnp.int32, (BLK, 1024), 1)
    k0 = ((g * BLK + row) * 1024 + col) * 4
    k0f = k0.astype(jnp.float32)
    total = out_ref[0, 0]
    for b in range(4):
        mb = (lax.shift_right_logical(w, 8 * b) & 0xFF) != 0
        total = total + jnp.sum(jnp.where(mb, k0f + float(b), 0.0))
    out_ref[0, 0] = total

    @pl.when(g == GRID - 1)
    def _():
        out_ref[0, 0] = out_ref[0, 0] * (2.0 ** -23)


def _tc_merge(p):
    p2 = p.reshape(ROWS, 1024)
    out = pl.pallas_call(
        _tc_merge_body,
        grid=(GRID,),
        in_specs=[pl.BlockSpec((BLK, 1024), lambda g: (g, 0))],
        out_specs=pl.BlockSpec(memory_space=pltpu.MemorySpace.SMEM),
        out_shape=jax.ShapeDtypeStruct((1, 1), jnp.float32),
    )(p2)
    return out.reshape(())


def kernel(x):
    counts = _sc_scatter(x)
    return _tc_merge(counts)
